# Initial kernel scaffold; baseline (speedup 1.0000x reference)
import jax
import jax.numpy as jnp
from jax.experimental import pallas as pl


def kernel(q, p, e, qv, params):
    raise NotImplementedError("write your pallas kernel here")


# speedup vs baseline: 8.4593x; 8.4593x over previous
import jax
import jax.numpy as jnp
import numpy as np
from jax.experimental import pallas as pl
from jax.experimental.pallas import tpu as pltpu

B_SZ = 4096
D = 768
H = 4
OC1 = 192
BT = 128

_EDGES = [(0, 1, 0), (1, 0, 0), (0, 2, 0), (2, 0, 0), (0, 3, 0), (3, 0, 0),
          (1, 2, 1), (2, 1, 1), (1, 3, 1), (3, 1, 1), (2, 3, 1), (3, 2, 1)]
_INC = {d: [(s, wk) for (s, dd, wk) in _EDGES if dd == d] for d in range(4)}


def _dot(a, b):
    return jax.lax.dot_general(a, b, (((1,), (0,)), ((), ())),
                               preferred_element_type=jnp.float32,
                               precision=jax.lax.Precision.HIGHEST)


def _body(q_ref, p_ref, e_ref, qv_ref, te_ref,
          wl1_ref, wr1_ref, a1_ref, b1m_ref,
          wl2_ref, wr2_ref, w1_ref, w2_ref,
          v768_ref, v3072_ref, out_ref):
    xs = [q_ref[...] + te_ref[0:1, :],
          p_ref[...] + te_ref[1:2, :],
          e_ref[...] + te_ref[2:3, :],
          qv_ref[...] + te_ref[3:4, :]]
    x = jnp.concatenate(xs, axis=0)

    bl1 = v768_ref[0:1, :]
    br1 = v768_ref[1:2, :]
    ea1 = [v768_ref[2:3, :], v768_ref[3:4, :]]
    bias1 = v768_ref[5:6, :]
    bias2 = v768_ref[6:7, :]
    b1v = v768_ref[7:8, :]
    ln_g = v768_ref[8:9, :]
    ln_b = v768_ref[9:10, :]
    b2v = v768_ref[10:11, :]

    xl = _dot(x, wl1_ref[...]) + bl1
    xr = _dot(x, wr1_ref[...]) + br1
    xln = [xl[n * BT:(n + 1) * BT] for n in range(4)]
    xrn = [xr[n * BT:(n + 1) * BT] for n in range(4)]

    x2n = []
    for dn in range(4):
        logits, srcs = [], []
        for (s, wk) in _INC[dn]:
            m = xln[s] + xrn[dn] + ea1[wk]
            m = jnp.where(m > 0, m, 0.2 * m)
            logits.append(_dot(m, a1_ref[...]))
            srcs.append(s)
        amax = jnp.maximum(jnp.maximum(logits[0], logits[1]), logits[2])
        exps = [jnp.exp(l - amax) for l in logits]
        den = exps[0] + exps[1] + exps[2] + 1e-16
        outd = None
        for ex, s in zip(exps, srcs):
            af = _dot(ex / den, b1m_ref[...])
            t = af * xln[s]
            outd = t if outd is None else outd + t
        r = xs[dn] + outd + bias1
        x2n.append(jnp.where(r > 0, r, jnp.exp(r) - 1.0))
    x2 = jnp.concatenate(x2n, axis=0)

    acc = [None] * 4
    for h in range(4):
        sl = slice(h * D, (h + 1) * D)
        bl2h = v3072_ref[0:1, sl]
        br2h = v3072_ref[1:2, sl]
        ea2h = [v3072_ref[2:3, sl], v3072_ref[3:4, sl]]
        att2h = v3072_ref[4:5, sl]
        xl2 = _dot(x2, wl2_ref[:, sl]) + bl2h
        xr2 = _dot(x2, wr2_ref[:, sl]) + br2h
        xl2n = [xl2[n * BT:(n + 1) * BT] for n in range(4)]
        xr2n = [xr2[n * BT:(n + 1) * BT] for n in range(4)]
        for dn in range(4):
            logits, srcs = [], []
            for (s, wk) in _INC[dn]:
                m = xl2n[s] + xr2n[dn] + ea2h[wk]
                m = jnp.where(m > 0, m, 0.2 * m)
                logits.append(jnp.sum(m * att2h, axis=1, keepdims=True))
                srcs.append(s)
            amax = jnp.maximum(jnp.maximum(logits[0], logits[1]), logits[2])
            exps = [jnp.exp(l - amax) for l in logits]
            den = exps[0] + exps[1] + exps[2] + 1e-16
            outd = None
            for ex, s in zip(exps, srcs):
                t = (ex / den) * xl2n[s]
                outd = t if outd is None else outd + t
            acc[dn] = outd if acc[dn] is None else acc[dn] + outd
    x3 = jnp.concatenate(
        [x2n[dn] + 0.25 * acc[dn] + bias2 for dn in range(4)], axis=0)

    hmid = _dot(x3, w1_ref[...]) + b1v
    mu = jnp.mean(hmid, axis=1, keepdims=True)
    var = jnp.mean((hmid - mu) ** 2, axis=1, keepdims=True)
    hmid = (hmid - mu) * jax.lax.rsqrt(var + 1e-5) * ln_g + ln_b
    hmid = jnp.maximum(hmid, 0.0)
    out = _dot(hmid, w2_ref[...]) + b2v
    for n in range(4):
        out_ref[n, :, :] = out[n * BT:(n + 1) * BT]


def kernel(q, p, e, qv, params):
    L = params['layers']
    w = jax.nn.sigmoid(params['weak_weight'])
    f32 = jnp.float32

    we1 = L[0]['We'].reshape(-1)
    att1 = L[0]['att'].reshape(-1)
    we2 = L[1]['We'].reshape(-1)
    att2 = L[1]['att'].reshape(-1)

    zeros = jnp.zeros((D,), f32)
    v768 = jnp.stack([
        L[0]['bl'], L[0]['br'], we1, w * we1, zeros,
        L[0]['bias'], L[1]['bias'], params['b1'], params['ln_g'],
        params['ln_b'], params['b2'], zeros, zeros, zeros, zeros, zeros,
    ]).astype(f32)
    z2 = jnp.zeros((4 * D,), f32)
    v3072 = jnp.stack([
        L[1]['bl'], L[1]['br'], we2, w * we2, att2, z2, z2, z2,
    ]).astype(f32)

    head_of_lane = jnp.arange(D) // OC1
    a1 = jnp.where(head_of_lane[:, None] == jnp.arange(128)[None, :],
                   att1[:, None], 0.0).astype(f32)
    b1m = (jnp.arange(128)[:, None] == head_of_lane[None, :]).astype(f32)

    cspec = lambda shape: pl.BlockSpec(shape, lambda i: (0,) * len(shape))
    out = pl.pallas_call(
        _body,
        grid=(B_SZ // BT,),
        in_specs=[
            pl.BlockSpec((BT, D), lambda i: (i, 0)),
            pl.BlockSpec((BT, D), lambda i: (i, 0)),
            pl.BlockSpec((BT, D), lambda i: (i, 0)),
            pl.BlockSpec((BT, D), lambda i: (i, 0)),
            cspec((4, D)),
            cspec((D, D)),
            cspec((D, D)),
            cspec((D, 128)),
            cspec((128, D)),
            cspec((D, 4 * D)),
            cspec((D, 4 * D)),
            cspec((D, D)),
            cspec((D, D)),
            cspec((16, D)),
            cspec((8, 4 * D)),
        ],
        out_specs=pl.BlockSpec((4, BT, D), lambda i: (0, i, 0)),
        out_shape=jax.ShapeDtypeStruct((4, B_SZ, D), f32),
    )(q, p, e, qv, params['type_embed'],
      L[0]['Wl'], L[0]['Wr'], a1, b1m,
      L[1]['Wl'], L[1]['Wr'], params['W1'], params['W2'],
      v768, v3072)
    return out.transpose(1, 0, 2)


# speedup vs baseline: 16.0086x; 1.8924x over previous
import jax
import jax.numpy as jnp
import numpy as np
from jax.experimental import pallas as pl
from jax.experimental.pallas import tpu as pltpu

B_SZ = 4096
D = 768
H = 4
OC1 = 192
BT = 128

_EDGES = [(0, 1, 0), (1, 0, 0), (0, 2, 0), (2, 0, 0), (0, 3, 0), (3, 0, 0),
          (1, 2, 1), (2, 1, 1), (1, 3, 1), (3, 1, 1), (2, 3, 1), (3, 2, 1)]
_INC = {d: [(s, wk) for (s, dd, wk) in _EDGES if dd == d] for d in range(4)}


def _bdot(a, b):
    return jax.lax.dot_general(a, b, (((1,), (0,)), ((), ())),
                               preferred_element_type=jnp.float32)


def _split(a):
    hi = a.astype(jnp.bfloat16)
    lo = (a - hi.astype(jnp.float32)).astype(jnp.bfloat16)
    return hi, lo


def _dot3(a, b_hi, b_lo):
    a_hi, a_lo = _split(a)
    return (_bdot(a_hi, b_hi) + _bdot(a_lo, b_hi)) + _bdot(a_hi, b_lo)


def _body(q_ref, p_ref, e_ref, qv_ref, te_ref,
          wlr1h_ref, wlr1l_ref, a1h_ref, a1l_ref, b1m_ref,
          wlr2h_ref, wlr2l_ref, w1h_ref, w1l_ref, w2h_ref, w2l_ref,
          v768_ref, v3072_ref, out_ref):
    xs = [q_ref[...] + te_ref[0:1, :],
          p_ref[...] + te_ref[1:2, :],
          e_ref[...] + te_ref[2:3, :],
          qv_ref[...] + te_ref[3:4, :]]
    x = jnp.concatenate(xs, axis=0)

    bl1 = v768_ref[0:1, :]
    br1 = v768_ref[1:2, :]
    ea1 = [v768_ref[2:3, :], v768_ref[3:4, :]]
    bias1 = v768_ref[5:6, :]
    bias2 = v768_ref[6:7, :]
    b1v = v768_ref[7:8, :]
    ln_g = v768_ref[8:9, :]
    ln_b = v768_ref[9:10, :]
    b2v = v768_ref[10:11, :]

    xlr = _dot3(x, wlr1h_ref[...], wlr1l_ref[...])
    xln = [xlr[n * BT:(n + 1) * BT, 0:D] + bl1 for n in range(4)]
    xrn = [xlr[n * BT:(n + 1) * BT, D:2 * D] + br1 for n in range(4)]

    x2n = []
    for dn in range(4):
        logits, srcs = [], []
        for (s, wk) in _INC[dn]:
            m = xln[s] + xrn[dn] + ea1[wk]
            m = jnp.where(m > 0, m, 0.2 * m)
            logits.append(_dot3(m, a1h_ref[...], a1l_ref[...]))
            srcs.append(s)
        amax = jnp.maximum(jnp.maximum(logits[0], logits[1]), logits[2])
        exps = [jnp.exp(l - amax) for l in logits]
        den = exps[0] + exps[1] + exps[2] + 1e-16
        outd = None
        for ex, s in zip(exps, srcs):
            a_hi, a_lo = _split(ex / den)
            af = _bdot(a_hi, b1m_ref[...]) + _bdot(a_lo, b1m_ref[...])
            t = af * xln[s]
            outd = t if outd is None else outd + t
        r = xs[dn] + outd + bias1
        x2n.append(jnp.where(r > 0, r, jnp.exp(r) - 1.0))
    x2 = jnp.concatenate(x2n, axis=0)

    xlr2 = _dot3(x2, wlr2h_ref[...], wlr2l_ref[...])
    acc = [None] * 4
    for h in range(4):
        o = h * D
        bl2h = v3072_ref[0:1, o:o + D]
        br2h = v3072_ref[1:2, o:o + D]
        ea2h = [v3072_ref[2:3, o:o + D], v3072_ref[3:4, o:o + D]]
        att2h = v3072_ref[4:5, o:o + D]
        xl2n = [xlr2[n * BT:(n + 1) * BT, o:o + D] + bl2h for n in range(4)]
        xr2n = [xlr2[n * BT:(n + 1) * BT, 4 * D + o:4 * D + o + D] + br2h
                for n in range(4)]
        for dn in range(4):
            logits, srcs = [], []
            for (s, wk) in _INC[dn]:
                m = xl2n[s] + xr2n[dn] + ea2h[wk]
                m = jnp.where(m > 0, m, 0.2 * m)
                logits.append(jnp.sum(m * att2h, axis=1, keepdims=True))
                srcs.append(s)
            amax = jnp.maximum(jnp.maximum(logits[0], logits[1]), logits[2])
            exps = [jnp.exp(l - amax) for l in logits]
            den = exps[0] + exps[1] + exps[2] + 1e-16
            outd = None
            for ex, s in zip(exps, srcs):
                t = (ex / den) * xl2n[s]
                outd = t if outd is None else outd + t
            acc[dn] = outd if acc[dn] is None else acc[dn] + outd
    x3 = jnp.concatenate(
        [x2n[dn] + 0.25 * acc[dn] + bias2 for dn in range(4)], axis=0)

    hmid = _dot3(x3, w1h_ref[...], w1l_ref[...]) + b1v
    mu = jnp.mean(hmid, axis=1, keepdims=True)
    var = jnp.mean((hmid - mu) ** 2, axis=1, keepdims=True)
    hmid = (hmid - mu) * jax.lax.rsqrt(var + 1e-5) * ln_g + ln_b
    hmid = jnp.maximum(hmid, 0.0)
    out = _dot3(hmid, w2h_ref[...], w2l_ref[...]) + b2v
    for n in range(4):
        out_ref[n, :, :] = out[n * BT:(n + 1) * BT]


def kernel(q, p, e, qv, params):
    L = params['layers']
    w = jax.nn.sigmoid(params['weak_weight'])
    f32 = jnp.float32

    we1 = L[0]['We'].reshape(-1)
    att1 = L[0]['att'].reshape(-1)
    we2 = L[1]['We'].reshape(-1)
    att2 = L[1]['att'].reshape(-1)

    zeros = jnp.zeros((D,), f32)
    v768 = jnp.stack([
        L[0]['bl'], L[0]['br'], we1, w * we1, zeros,
        L[0]['bias'], L[1]['bias'], params['b1'], params['ln_g'],
        params['ln_b'], params['b2'], zeros, zeros, zeros, zeros, zeros,
    ]).astype(f32)
    z2 = jnp.zeros((4 * D,), f32)
    v3072 = jnp.stack([
        L[1]['bl'], L[1]['br'], we2, w * we2, att2, z2, z2, z2,
    ]).astype(f32)

    head_of_lane = jnp.arange(D) // OC1
    a1 = jnp.where(head_of_lane[:, None] == jnp.arange(128)[None, :],
                   att1[:, None], 0.0).astype(f32)
    b1m = (jnp.arange(128)[:, None] == head_of_lane[None, :]).astype(
        jnp.bfloat16)

    wlr1 = jnp.concatenate([L[0]['Wl'], L[0]['Wr']], axis=1)
    wlr2 = jnp.concatenate([L[1]['Wl'], L[1]['Wr']], axis=1)
    wlr1h, wlr1l = _split(wlr1)
    wlr2h, wlr2l = _split(wlr2)
    a1h, a1l = _split(a1)
    w1h, w1l = _split(params['W1'])
    w2h, w2l = _split(params['W2'])

    cspec = lambda shape: pl.BlockSpec(shape, lambda i: (0,) * len(shape))
    out = pl.pallas_call(
        _body,
        grid=(B_SZ // BT,),
        in_specs=[
            pl.BlockSpec((BT, D), lambda i: (i, 0)),
            pl.BlockSpec((BT, D), lambda i: (i, 0)),
            pl.BlockSpec((BT, D), lambda i: (i, 0)),
            pl.BlockSpec((BT, D), lambda i: (i, 0)),
            cspec((4, D)),
            cspec((D, 2 * D)),
            cspec((D, 2 * D)),
            cspec((D, 128)),
            cspec((D, 128)),
            cspec((128, D)),
            cspec((D, 8 * D)),
            cspec((D, 8 * D)),
            cspec((D, D)),
            cspec((D, D)),
            cspec((D, D)),
            cspec((D, D)),
            cspec((16, D)),
            cspec((8, 4 * D)),
        ],
        out_specs=pl.BlockSpec((4, BT, D), lambda i: (0, i, 0)),
        out_shape=jax.ShapeDtypeStruct((4, B_SZ, D), f32),
    )(q, p, e, qv, params['type_embed'],
      wlr1h, wlr1l, a1h, a1l, b1m,
      wlr2h, wlr2l, w1h, w1l, w2h, w2l,
      v768, v3072)
    return out.transpose(1, 0, 2)
